# baseline (speedup 1.0000x reference)
import jax
import jax.numpy as jnp
from jax.experimental import pallas as pl
from jax.experimental.pallas import tpu as pltpu

_N = 16777216
_R, _C = _N // 128, 128
_G = 32
_CR = _R // _G
_RESCH = 24
_SLOTS = _RESCH + 3
_SUB = 256


def _slot(j):
    if isinstance(j, int):
        return j if j < _RESCH else _RESCH + (j % 3)
    return jnp.where(j < _RESCH, j, _RESCH + (j % 3))


def _copy(x_hbm, buf, sems, j):
    return pltpu.make_async_copy(
        x_hbm.at[pl.ds(j * _CR, _CR), :],
        buf.at[pl.ds(_slot(j) * _CR, _CR), :],
        sems.at[_slot(j)],
    )


def _body(denom_ref, x_hbm, o_ref, buf, acc, sems):
    i = pl.program_id(0)

    @pl.when(i == 0)
    def _():
        for j in range(_RESCH + 3):
            _copy(x_hbm, buf, sems, j).start()

    @pl.when(i < _G)
    def _():
        _copy(x_hbm, buf, sems, i).wait()
        base = _slot(i) * _CR

        def _red(k, carry):
            cmn, cmx = carry
            v = buf[pl.ds(base + k * _SUB, _SUB), :]
            return jnp.minimum(cmn, v), jnp.maximum(cmx, v)

        v0 = buf[pl.ds(base, _SUB), :]
        cmn, cmx = jax.lax.fori_loop(1, _CR // _SUB, _red, (v0, v0))
        bmin = jnp.min(cmn)
        bmax = jnp.max(cmx)

        @pl.when(i == 0)
        def _():
            acc[0] = bmin
            acc[1] = bmax

        @pl.when(i > 0)
        def _():
            acc[0] = jnp.minimum(acc[0], bmin)
            acc[1] = jnp.maximum(acc[1], bmax)

        if _RESCH + 3 < _G:

            @pl.when(jnp.logical_and(i >= _RESCH, i + 3 < _G))
            def _():
                _copy(x_hbm, buf, sems, i + 3).start()

    @pl.when(i >= _G)
    def _():
        j = i - _G

        @pl.when(j == 0)
        def _():
            for jj in range(_RESCH, min(_RESCH + 3, _G)):
                _copy(x_hbm, buf, sems, jj).start()

        @pl.when(j >= _RESCH)
        def _():
            _copy(x_hbm, buf, sems, j).wait()

        mn = acc[0]
        sc = (acc[1] - mn) / denom_ref[0]
        inv = 1.0 / sc
        base = _slot(j) * _CR

        def _quant(k, carry):
            v = buf[pl.ds(base + k * _SUB, _SUB), :]
            o_ref[pl.ds(k * _SUB, _SUB), :] = (
                jnp.round((v - mn) * inv) * sc + mn
            )
            return carry

        jax.lax.fori_loop(0, _CR // _SUB, _quant, 0)

        @pl.when(jnp.logical_and(j >= _RESCH, j + 3 < _G))
        def _():
            _copy(x_hbm, buf, sems, j + 3).start()


def kernel(tensor, bits):
    x = tensor.reshape(_R, _C)
    denom = jnp.asarray((2 ** bits) - 1, dtype=jnp.float32).reshape(1)

    y = pl.pallas_call(
        _body,
        grid=(2 * _G,),
        in_specs=[
            pl.BlockSpec(memory_space=pltpu.SMEM),
            pl.BlockSpec(memory_space=pl.ANY),
        ],
        out_specs=pl.BlockSpec(
            (_CR, _C), lambda i: (jnp.where(i < _G, 0, i - _G), 0)
        ),
        out_shape=jax.ShapeDtypeStruct((_R, _C), jnp.float32),
        scratch_shapes=[
            pltpu.VMEM((_SLOTS * _CR, _C), jnp.float32),
            pltpu.SMEM((2,), jnp.float32),
            pltpu.SemaphoreType.DMA((_SLOTS,)),
        ],
    )(denom, x)

    return y.reshape(tensor.shape)


# speedup vs baseline: 1.0257x; 1.0257x over previous
import jax
import jax.numpy as jnp
from jax.experimental import pallas as pl
from jax.experimental.pallas import tpu as pltpu

_N = 16777216
_R, _C = _N // 128, 128
_G = 32
_CR = _R // _G
_RESCH = 24
_SLOTS = _RESCH + 3
_SUB = 128


def _slot(j):
    if isinstance(j, int):
        return j if j < _RESCH else _RESCH + (j % 3)
    return jnp.where(j < _RESCH, j, _RESCH + (j % 3))


def _copy(x_hbm, buf, sems, j):
    return pltpu.make_async_copy(
        x_hbm.at[pl.ds(j * _CR, _CR), :],
        buf.at[pl.ds(_slot(j) * _CR, _CR), :],
        sems.at[_slot(j)],
    )


def _body(denom_ref, x_hbm, o_ref, buf, acc, sems):
    i = pl.program_id(0)

    @pl.when(i == 0)
    def _():
        for j in range(_RESCH + 3):
            _copy(x_hbm, buf, sems, j).start()

    @pl.when(i < _G)
    def _():
        _copy(x_hbm, buf, sems, i).wait()
        base = _slot(i) * _CR

        def _red(k, carry):
            cmn, cmx = carry
            v = buf[pl.ds(base + k * _SUB, _SUB), :]
            return jnp.minimum(cmn, v), jnp.maximum(cmx, v)

        v0 = buf[pl.ds(base, _SUB), :]
        cmn, cmx = jax.lax.fori_loop(1, _CR // _SUB, _red, (v0, v0))
        bmin = jnp.min(cmn)
        bmax = jnp.max(cmx)

        @pl.when(i == 0)
        def _():
            acc[0] = bmin
            acc[1] = bmax

        @pl.when(i > 0)
        def _():
            acc[0] = jnp.minimum(acc[0], bmin)
            acc[1] = jnp.maximum(acc[1], bmax)

        if _RESCH + 3 < _G:

            @pl.when(jnp.logical_and(i >= _RESCH, i + 3 < _G))
            def _():
                _copy(x_hbm, buf, sems, i + 3).start()

    @pl.when(i >= _G)
    def _():
        j = i - _G

        @pl.when(j == 0)
        def _():
            for jj in range(_RESCH, min(_RESCH + 3, _G)):
                _copy(x_hbm, buf, sems, jj).start()

        @pl.when(j >= _RESCH)
        def _():
            _copy(x_hbm, buf, sems, j).wait()

        mn = acc[0]
        sc = (acc[1] - mn) / denom_ref[0]
        inv = 1.0 / sc
        base = _slot(j) * _CR

        def _quant(k, carry):
            v = buf[pl.ds(base + k * _SUB, _SUB), :]
            o_ref[pl.ds(k * _SUB, _SUB), :] = (
                jnp.round((v - mn) * inv) * sc + mn
            )
            return carry

        jax.lax.fori_loop(0, _CR // _SUB, _quant, 0)

        @pl.when(jnp.logical_and(j >= _RESCH, j + 3 < _G))
        def _():
            _copy(x_hbm, buf, sems, j + 3).start()


def kernel(tensor, bits):
    x = tensor.reshape(_R, _C)
    denom = jnp.asarray((2 ** bits) - 1, dtype=jnp.float32).reshape(1)

    y = pl.pallas_call(
        _body,
        grid=(2 * _G,),
        in_specs=[
            pl.BlockSpec(memory_space=pltpu.SMEM),
            pl.BlockSpec(memory_space=pl.ANY),
        ],
        out_specs=pl.BlockSpec(
            (_CR, _C), lambda i: (jnp.where(i < _G, 0, i - _G), 0)
        ),
        out_shape=jax.ShapeDtypeStruct((_R, _C), jnp.float32),
        scratch_shapes=[
            pltpu.VMEM((_SLOTS * _CR, _C), jnp.float32),
            pltpu.SMEM((2,), jnp.float32),
            pltpu.SemaphoreType.DMA((_SLOTS,)),
        ],
    )(denom, x)

    return y.reshape(tensor.shape)


# speedup vs baseline: 1.1962x; 1.1662x over previous
import jax
import jax.numpy as jnp
from jax.experimental import pallas as pl
from jax.experimental.pallas import tpu as pltpu

_N = 16777216
_R, _C = _N // 128, 128
_G = 32
_CR = _R // _G
_RESCH = 25
_ROT = 4
_SLOTS = _RESCH + _ROT
_SUB = 128


def _slot(j):
    if isinstance(j, int):
        return j if j < _RESCH else _RESCH + (j % _ROT)
    return jnp.where(j < _RESCH, j, _RESCH + (j % _ROT))


def _fetch(x_hbm, buf, sems, j):
    return pltpu.make_async_copy(
        x_hbm.at[pl.ds(j * _CR, _CR), :],
        buf.at[pl.ds(_slot(j) * _CR, _CR), :],
        sems.at[_slot(j)],
    )


def _put(o_hbm, buf, sems, j):
    return pltpu.make_async_copy(
        buf.at[pl.ds(_slot(j) * _CR, _CR), :],
        o_hbm.at[pl.ds(j * _CR, _CR), :],
        sems.at[_slot(j)],
    )


def _body(denom_ref, x_hbm, o_hbm, buf, in_sems, out_sems):
    for j in range(_SLOTS):
        _fetch(x_hbm, buf, in_sems, j).start()

    def _chunk_red(i, carry):
        _fetch(x_hbm, buf, in_sems, i).wait()
        base = _slot(i) * _CR

        def _red(k, c2):
            a, b = c2
            v = buf[pl.ds(base + k * _SUB, _SUB), :]
            return jnp.minimum(a, v), jnp.maximum(b, v)

        carry = jax.lax.fori_loop(0, _CR // _SUB, _red, carry)

        @pl.when(jnp.logical_and(i + _ROT >= _SLOTS, i + _ROT < _G))
        def _():
            _fetch(x_hbm, buf, in_sems, i + _ROT).start()

        return carry

    inf = jnp.float32(jnp.inf)
    cmn, cmx = jax.lax.fori_loop(
        0,
        _G,
        _chunk_red,
        (jnp.full((_SUB, _C), inf), jnp.full((_SUB, _C), -inf)),
    )
    mn = jnp.min(cmn)
    sc = (jnp.max(cmx) - mn) / denom_ref[0]
    inv = 1.0 / sc

    for j in range(_RESCH, min(_RESCH + _ROT, _G)):
        _fetch(x_hbm, buf, in_sems, j).start()

    def _chunk_q(j, carry):
        @pl.when(j >= _RESCH)
        def _():
            _fetch(x_hbm, buf, in_sems, j).wait()

        base = _slot(j) * _CR

        def _quant(k, c2):
            r = pl.ds(base + k * _SUB, _SUB)
            buf[r, :] = jnp.round((buf[r, :] - mn) * inv) * sc + mn
            return c2

        jax.lax.fori_loop(0, _CR // _SUB, _quant, 0)
        _put(o_hbm, buf, out_sems, j).start()

        @pl.when(jnp.logical_and(j + 2 >= _RESCH + _ROT, j + 2 < _G))
        def _():
            _put(o_hbm, buf, out_sems, j + 2).wait()
            _fetch(x_hbm, buf, in_sems, j + 2).start()

        return carry

    jax.lax.fori_loop(0, _G, _chunk_q, 0)

    for s in range(_SLOTS):
        pltpu.make_async_copy(
            buf.at[pl.ds(s * _CR, _CR), :],
            o_hbm.at[pl.ds(s * _CR, _CR), :],
            out_sems.at[s],
        ).wait()


def kernel(tensor, bits):
    x = tensor.reshape(_R, _C)
    denom = jnp.asarray((2 ** bits) - 1, dtype=jnp.float32).reshape(1)

    y = pl.pallas_call(
        _body,
        in_specs=[
            pl.BlockSpec(memory_space=pltpu.SMEM),
            pl.BlockSpec(memory_space=pl.ANY),
        ],
        out_specs=pl.BlockSpec(memory_space=pl.ANY),
        out_shape=jax.ShapeDtypeStruct((_R, _C), jnp.float32),
        scratch_shapes=[
            pltpu.VMEM((_SLOTS * _CR, _C), jnp.float32),
            pltpu.SemaphoreType.DMA((_SLOTS,)),
            pltpu.SemaphoreType.DMA((_SLOTS,)),
        ],
    )(denom, x)

    return y.reshape(tensor.shape)


# speedup vs baseline: 1.2930x; 1.0809x over previous
import jax
import jax.numpy as jnp
from jax.experimental import pallas as pl
from jax.experimental.pallas import tpu as pltpu

_N = 16777216
_R, _C = _N // 128, 128
_G = 32
_CR = _R // _G
_RESCH = 25
_ROT = 4
_SLOTS = _RESCH + _ROT
_REF = _G - _SLOTS
_SUB = 128


def _slot(j):
    if isinstance(j, int):
        return j if j < _RESCH else _RESCH + (j % _ROT)
    return jnp.where(j < _RESCH, j, _RESCH + (j % _ROT))


def _fetch(x_hbm, buf, sems, j):
    return pltpu.make_async_copy(
        x_hbm.at[pl.ds(j * _CR, _CR), :],
        buf.at[pl.ds(_slot(j) * _CR, _CR), :],
        sems.at[_slot(j)],
    )


def _put(o_hbm, buf, sems, j):
    return pltpu.make_async_copy(
        buf.at[pl.ds(_slot(j) * _CR, _CR), :],
        o_hbm.at[pl.ds(j * _CR, _CR), :],
        sems.at[_slot(j)],
    )


def _body(denom_ref, x_hbm, o_hbm, buf, in_sems, out_sems):
    for j in range(_SLOTS):
        _fetch(x_hbm, buf, in_sems, j).start()

    def _chunk_red(i, carry):
        _fetch(x_hbm, buf, in_sems, i).wait()
        base = _slot(i) * _CR

        def _red(k, c2):
            a, b = c2
            v = buf[pl.ds(base + k * _SUB, _SUB), :]
            return jnp.minimum(a, v), jnp.maximum(b, v)

        carry = jax.lax.fori_loop(0, _CR // _SUB, _red, carry)

        @pl.when(jnp.logical_and(i + _ROT >= _SLOTS, i + _ROT < _G))
        def _():
            _fetch(x_hbm, buf, in_sems, i + _ROT).start()

        return carry

    inf = jnp.float32(jnp.inf)
    cmn, cmx = jax.lax.fori_loop(
        0,
        _G,
        _chunk_red,
        (jnp.full((_SUB, _C), inf), jnp.full((_SUB, _C), -inf)),
    )
    mn = jnp.min(cmn)
    sc = (jnp.max(cmx) - mn) / denom_ref[0]
    inv = 1.0 / sc

    def _chunk_q(j, carry):
        cj = jnp.where(
            j < _RESCH, j, jnp.where(j < _RESCH + _ROT, j + _REF, j - _ROT)
        )
        sj = jnp.where(
            j < _RESCH,
            j,
            jnp.where(
                j < _RESCH + _ROT,
                _RESCH + ((j + _REF) % _ROT),
                j - (_RESCH + _ROT),
            ),
        )

        @pl.when(j >= _RESCH + _ROT)
        def _():
            pltpu.make_async_copy(
                x_hbm.at[pl.ds(cj * _CR, _CR), :],
                buf.at[pl.ds(sj * _CR, _CR), :],
                in_sems.at[sj],
            ).wait()

        base = sj * _CR

        def _quant(k, c2):
            r = pl.ds(base + k * _SUB, _SUB)
            buf[r, :] = jnp.round((buf[r, :] - mn) * inv) * sc + mn
            return c2

        jax.lax.fori_loop(0, _CR // _SUB, _quant, 0)
        pltpu.make_async_copy(
            buf.at[pl.ds(sj * _CR, _CR), :],
            o_hbm.at[pl.ds(cj * _CR, _CR), :],
            out_sems.at[sj],
        ).start()

        @pl.when(jnp.logical_and(j >= _REF, j < 2 * _REF))
        def _():
            s = j - _REF
            pltpu.make_async_copy(
                buf.at[pl.ds(s * _CR, _CR), :],
                o_hbm.at[pl.ds(s * _CR, _CR), :],
                out_sems.at[s],
            ).wait()
            pltpu.make_async_copy(
                x_hbm.at[pl.ds((_RESCH + s) * _CR, _CR), :],
                buf.at[pl.ds(s * _CR, _CR), :],
                in_sems.at[s],
            ).start()

        return carry

    jax.lax.fori_loop(0, _G, _chunk_q, 0)

    for s in range(_SLOTS):
        pltpu.make_async_copy(
            buf.at[pl.ds(s * _CR, _CR), :],
            o_hbm.at[pl.ds(s * _CR, _CR), :],
            out_sems.at[s],
        ).wait()


def kernel(tensor, bits):
    x = tensor.reshape(_R, _C)
    denom = jnp.asarray((2 ** bits) - 1, dtype=jnp.float32).reshape(1)

    y = pl.pallas_call(
        _body,
        in_specs=[
            pl.BlockSpec(memory_space=pltpu.SMEM),
            pl.BlockSpec(memory_space=pl.ANY),
        ],
        out_specs=pl.BlockSpec(memory_space=pl.ANY),
        out_shape=jax.ShapeDtypeStruct((_R, _C), jnp.float32),
        scratch_shapes=[
            pltpu.VMEM((_SLOTS * _CR, _C), jnp.float32),
            pltpu.SemaphoreType.DMA((_SLOTS,)),
            pltpu.SemaphoreType.DMA((_SLOTS,)),
        ],
    )(denom, x)

    return y.reshape(tensor.shape)
